# baseline (speedup 1.0000x reference)
import functools

import jax
import jax.numpy as jnp
from jax import lax
from jax.experimental import pallas as pl
from jax.experimental.pallas import tpu as pltpu
from jax.experimental.pallas import tpu_sc as plsc

_B = 16384


@functools.cache
def _build_gather():
    info = plsc.get_sparse_core_info()
    nc, ns = info.num_cores, info.num_subcores
    nc = 1
    nw = nc * ns
    b_per_w = _B // nw

    mesh = plsc.VectorSubcoreMesh(
        core_axis_name="c", subcore_axis_name="s", num_cores=nc)

    half = b_per_w // 2

    @functools.partial(
        pl.kernel,
        mesh=mesh,
        out_type=jax.ShapeDtypeStruct((_B,), jnp.float32),
        scratch_types=[
            pltpu.VMEM((half,), jnp.int32),
            pltpu.VMEM((half,), jnp.int32),
            pltpu.VMEM((half,), jnp.float32),
            pltpu.VMEM((half,), jnp.float32),
            pltpu.SemaphoreType.DMA,
            pltpu.SemaphoreType.DMA,
            pltpu.SemaphoreType.DMA,
            pltpu.SemaphoreType.DMA,
            pltpu.SemaphoreType.DMA,
            pltpu.SemaphoreType.DMA,
        ],
    )
    def gather_kernel(preds_hbm, idx_hbm, out_hbm, idx0, idx1, val0, val1,
                      si0, si1, sg0, sg1, so0, so1):
        wid = lax.axis_index("s") * nc + lax.axis_index("c")
        base = wid * b_per_w
        i0 = pltpu.async_copy(idx_hbm.at[pl.ds(base, half)], idx0, si0)
        i1 = pltpu.async_copy(idx_hbm.at[pl.ds(base + half, half)], idx1, si1)
        i0.wait()
        g0 = pltpu.async_copy(preds_hbm.at[idx0], val0, sg0)
        i1.wait()
        g1 = pltpu.async_copy(preds_hbm.at[idx1], val1, sg1)
        g0.wait()
        o0 = pltpu.async_copy(val0, out_hbm.at[pl.ds(base, half)], so0)
        g1.wait()
        o1 = pltpu.async_copy(val1, out_hbm.at[pl.ds(base + half, half)], so1)
        o0.wait()
        o1.wait()

    return gather_kernel


def kernel(preds, y):
    out = _build_gather()(preds, y.astype(jnp.int32))
    return out.reshape(-1, 1)


# speedup vs baseline: 1.0012x; 1.0012x over previous
import functools

import jax
import jax.numpy as jnp
from jax import lax
from jax.experimental import pallas as pl
from jax.experimental.pallas import tpu as pltpu
from jax.experimental.pallas import tpu_sc as plsc

_B = 16384


@functools.cache
def _build_gather():
    info = plsc.get_sparse_core_info()
    nc, ns = info.num_cores, info.num_subcores
    nc = 1
    nw = nc * ns
    b_per_w = _B // nw

    mesh = plsc.VectorSubcoreMesh(
        core_axis_name="c", subcore_axis_name="s", num_cores=nc)

    nch = 4
    ch = b_per_w // nch

    @functools.partial(
        pl.kernel,
        mesh=mesh,
        out_type=jax.ShapeDtypeStruct((_B,), jnp.float32),
        scratch_types=[
            [pltpu.VMEM((ch,), jnp.int32)] * nch,
            [pltpu.VMEM((ch,), jnp.float32)] * nch,
            [pltpu.SemaphoreType.DMA] * nch,
            [pltpu.SemaphoreType.DMA] * nch,
            [pltpu.SemaphoreType.DMA] * nch,
        ],
    )
    def gather_kernel(preds_hbm, idx_hbm, out_hbm, idxs, vals, sis, sgs, sos):
        wid = lax.axis_index("s") * nc + lax.axis_index("c")
        base = wid * b_per_w
        iops = [
            pltpu.async_copy(idx_hbm.at[pl.ds(base + j * ch, ch)], idxs[j], sis[j])
            for j in range(nch)
        ]
        gops = []
        for j in range(nch):
            iops[j].wait()
            gops.append(pltpu.async_copy(preds_hbm.at[idxs[j]], vals[j], sgs[j]))
        oops = []
        for j in range(nch):
            gops[j].wait()
            oops.append(pltpu.async_copy(
                vals[j], out_hbm.at[pl.ds(base + j * ch, ch)], sos[j]))
        for o in oops:
            o.wait()

    return gather_kernel


def kernel(preds, y):
    out = _build_gather()(preds, y.astype(jnp.int32))
    return out.reshape(-1, 1)
